# baseline (speedup 1.0000x reference)
import functools
import math

import jax
import jax.numpy as jnp
from jax import lax
from jax.experimental import pallas as pl
from jax.experimental.pallas import tpu as pltpu
from jax.experimental.pallas import tpu_sc as plsc

D_MODEL = 128
SCALE = math.sqrt(float(D_MODEL))
NW = 32
NROW = 4096
NCOL = 50
C = NROW // NW
LANES = 16
NIN = 3
NOUT = 2


def _body(xt_hbm, lut_hbm, out_hbm, idx_v, in0, in1, in2, out0, out1,
          sg0, sg1, sg2, ss0, ss1):
    wid = lax.axis_index("s") * 2 + lax.axis_index("c")
    i0 = wid * C
    pltpu.sync_copy(xt_hbm.at[:, pl.ds(i0, C)], idx_v)

    ins = (in0, in1, in2)
    outs = (out0, out1)
    sgs = (sg0, sg1, sg2)
    sss = (ss0, ss1)

    def scale_chunk(src, dst):
        def row_body(r, carry):
            for c8 in range(D_MODEL // LANES):
                sl = pl.ds(c8 * LANES, LANES)
                dst[r, sl] = src[r, sl] * SCALE
            return carry

        lax.fori_loop(0, C, row_body, 0)

    def pipe_step(c, k, wait_out, issue_gather):
        bi = k % NIN
        bo = k % NOUT
        pltpu.make_async_copy(lut_hbm.at[idx_v.at[c]], ins[bi],
                              sgs[bi]).wait()
        if wait_out:
            pltpu.make_async_copy(
                outs[bo], out_hbm.at[c - NOUT, pl.ds(i0, C)], sss[bo]).wait()
        scale_chunk(ins[bi], outs[bo])
        pltpu.async_copy(outs[bo], out_hbm.at[c, pl.ds(i0, C)], sss[bo])
        if issue_gather:
            pltpu.async_copy(lut_hbm.at[idx_v.at[c + NIN]], ins[bi], sgs[bi])

    for k in range(NIN):
        pltpu.async_copy(lut_hbm.at[idx_v.at[k]], ins[k], sgs[k])

    for k in range(6):
        pipe_step(k, k, wait_out=(k >= NOUT), issue_gather=True)

    def loop_body(i, carry):
        for k in range(6):
            pipe_step(6 * i + k, k, wait_out=True, issue_gather=True)
        return carry

    lax.fori_loop(1, 7, loop_body, 0)

    for c in range(42, NCOL):
        pipe_step(c, c % 6, wait_out=True, issue_gather=(c + NIN < NCOL))

    for k in range(NOUT):
        c = NCOL - NOUT + k
        pltpu.make_async_copy(
            outs[c % NOUT], out_hbm.at[c, pl.ds(i0, C)], sss[c % NOUT]).wait()


_lookup = functools.partial(
    pl.kernel,
    out_type=jax.ShapeDtypeStruct((NCOL, NROW, D_MODEL), jnp.float32),
    scratch_types=[
        pltpu.VMEM((NCOL, C), jnp.int32),
        pltpu.VMEM((C, D_MODEL), jnp.float32),
        pltpu.VMEM((C, D_MODEL), jnp.float32),
        pltpu.VMEM((C, D_MODEL), jnp.float32),
        pltpu.VMEM((C, D_MODEL), jnp.float32),
        pltpu.VMEM((C, D_MODEL), jnp.float32),
        pltpu.SemaphoreType.DMA,
        pltpu.SemaphoreType.DMA,
        pltpu.SemaphoreType.DMA,
        pltpu.SemaphoreType.DMA,
        pltpu.SemaphoreType.DMA,
    ],
    mesh=plsc.VectorSubcoreMesh(core_axis_name="c", subcore_axis_name="s"),
    compiler_params=pltpu.CompilerParams(
        use_tc_tiling_on_sc=True,
        disable_bounds_checks=True,
        disable_semaphore_checks=True,
    ),
)(_body)


def kernel(x, lut):
    out = _lookup(x.T, lut)
    return out.transpose(1, 0, 2)


# speedup vs baseline: 1.0170x; 1.0170x over previous
import functools
import math

import jax
import jax.numpy as jnp
from jax import lax
from jax.experimental import pallas as pl
from jax.experimental.pallas import tpu as pltpu
from jax.experimental.pallas import tpu_sc as plsc

D_MODEL = 128
SCALE = math.sqrt(float(D_MODEL))
NW = 32
NROW = 4096
NCOL = 50
C = NROW // NW
LANES = 16
NIN = 4
NOUT = 2


def _body(xt_hbm, lut_hbm, out_hbm, idx_v, in0, in1, in2, in3, out0, out1,
          sg0, sg1, sg2, sg3, ss0, ss1):
    wid = lax.axis_index("s") * 2 + lax.axis_index("c")
    i0 = wid * C

    ins = (in0, in1, in2, in3)
    outs = (out0, out1)
    sgs = (sg0, sg1, sg2, sg3)
    sss = (ss0, ss1)

    pltpu.sync_copy(xt_hbm.at[pl.ds(0, 8), pl.ds(i0, C)],
                    idx_v.at[pl.ds(0, 8)])
    for k in range(NIN):
        pltpu.async_copy(lut_hbm.at[idx_v.at[k]], ins[k], sgs[k])
    pltpu.sync_copy(xt_hbm.at[pl.ds(8, NCOL - 8), pl.ds(i0, C)],
                    idx_v.at[pl.ds(8, NCOL - 8)])

    def scale_chunk(src, dst):
        def row_body(r, carry):
            for c8 in range(D_MODEL // LANES):
                sl = pl.ds(c8 * LANES, LANES)
                dst[r, sl] = src[r, sl] * SCALE
            return carry

        lax.fori_loop(0, C, row_body, 0)

    def pipe_step(c, k, wait_out, issue_gather):
        bi = k % NIN
        bo = k % NOUT
        pltpu.make_async_copy(lut_hbm.at[idx_v.at[c]], ins[bi],
                              sgs[bi]).wait()
        if wait_out:
            pltpu.make_async_copy(
                outs[bo], out_hbm.at[c - NOUT, pl.ds(i0, C)], sss[bo]).wait()
        scale_chunk(ins[bi], outs[bo])
        pltpu.async_copy(outs[bo], out_hbm.at[c, pl.ds(i0, C)], sss[bo])
        if issue_gather:
            pltpu.async_copy(lut_hbm.at[idx_v.at[c + NIN]], ins[bi], sgs[bi])

    for k in range(NIN):
        pipe_step(k, k, wait_out=(k >= NOUT), issue_gather=True)

    def loop_body(i, carry):
        for k in range(NIN):
            pipe_step(NIN * i + k, k, wait_out=True, issue_gather=True)
        return carry

    lax.fori_loop(1, 11, loop_body, 0)

    for c in range(44, NCOL):
        pipe_step(c, c % NIN, wait_out=True, issue_gather=(c + NIN < NCOL))

    for k in range(NOUT):
        c = NCOL - NOUT + k
        pltpu.make_async_copy(
            outs[c % NOUT], out_hbm.at[c, pl.ds(i0, C)], sss[c % NOUT]).wait()


_lookup = functools.partial(
    pl.kernel,
    out_type=jax.ShapeDtypeStruct((NCOL, NROW, D_MODEL), jnp.float32),
    scratch_types=[
        pltpu.VMEM((NCOL, C), jnp.int32),
        pltpu.VMEM((C, D_MODEL), jnp.float32),
        pltpu.VMEM((C, D_MODEL), jnp.float32),
        pltpu.VMEM((C, D_MODEL), jnp.float32),
        pltpu.VMEM((C, D_MODEL), jnp.float32),
        pltpu.VMEM((C, D_MODEL), jnp.float32),
        pltpu.VMEM((C, D_MODEL), jnp.float32),
        pltpu.SemaphoreType.DMA,
        pltpu.SemaphoreType.DMA,
        pltpu.SemaphoreType.DMA,
        pltpu.SemaphoreType.DMA,
        pltpu.SemaphoreType.DMA,
        pltpu.SemaphoreType.DMA,
    ],
    mesh=plsc.VectorSubcoreMesh(core_axis_name="c", subcore_axis_name="s"),
    compiler_params=pltpu.CompilerParams(use_tc_tiling_on_sc=True),
)(_body)


def kernel(x, lut):
    out = _lookup(x.T, lut)
    return out.transpose(1, 0, 2)
